# Initial kernel scaffold; baseline (speedup 1.0000x reference)
import functools

import jax
import jax.numpy as jnp
from jax import lax
from jax.experimental import pallas as pl
from jax.experimental.pallas import tpu as pltpu
from jax.experimental.pallas import tpu_sc as plsc

D = 1000
NC, NS = 2, 16
NW = NC * NS
W = 64


@functools.partial(jax.jit, static_argnames=("B",))
def _gather_rows(table, idx_flat, B):
    b_per_w = B // NW
    n_chunks = b_per_w // W
    mesh = plsc.VectorSubcoreMesh(core_axis_name="c", subcore_axis_name="s")

    @functools.partial(
        pl.kernel,
        mesh=mesh,
        out_type=jax.ShapeDtypeStruct((B, D), jnp.float32),
        scratch_types=[
            pltpu.VMEM((b_per_w,), jnp.int32),
            pltpu.VMEM((W, D), jnp.float32),
            pltpu.SemaphoreType.DMA,
        ],
    )
    def k(table_hbm, idx_hbm, out_hbm, idx_v, rows_v, sem):
        wid = lax.axis_index("s") * NC + lax.axis_index("c")
        base = wid * b_per_w
        pltpu.sync_copy(idx_hbm.at[pl.ds(base, b_per_w)], idx_v)

        @pl.loop(0, n_chunks)
        def _(c):
            off = c * W
            pltpu.async_copy(
                table_hbm.at[idx_v.at[pl.ds(off, W)]], rows_v, sem
            ).wait()
            pltpu.sync_copy(rows_v, out_hbm.at[pl.ds(base + off, W)])

    return k(table, idx_flat)


def kernel(idx, token_emb):
    B_, T_ = idx.shape
    B = B_ * T_
    idx_flat = idx.reshape(B).astype(jnp.int32)
    out = _gather_rows(token_emb, idx_flat, B)
    return out.reshape(B_, T_, D)


# speedup vs baseline: 1.4249x; 1.4249x over previous
import functools

import jax
import jax.numpy as jnp
from jax import lax
from jax.experimental import pallas as pl
from jax.experimental.pallas import tpu as pltpu
from jax.experimental.pallas import tpu_sc as plsc

D = 1000
DP = 1024
NC, NS = 2, 16
NW = NC * NS
W = 64


@functools.partial(jax.jit, static_argnames=("B",))
def _gather_rows(table, idx_flat, B):
    b_per_w = B // NW
    n_chunks = b_per_w // W
    mesh = plsc.VectorSubcoreMesh(core_axis_name="c", subcore_axis_name="s")

    @functools.partial(
        pl.kernel,
        mesh=mesh,
        out_type=jax.ShapeDtypeStruct((B, DP), jnp.float32),
        scratch_types=[
            pltpu.VMEM((b_per_w,), jnp.int32),
            pltpu.VMEM((W, DP), jnp.float32),
            pltpu.SemaphoreType.DMA,
        ],
    )
    def k(table_hbm, idx_hbm, out_hbm, idx_v, rows_v, sem):
        wid = lax.axis_index("s") * NC + lax.axis_index("c")
        base = wid * b_per_w
        pltpu.sync_copy(idx_hbm.at[pl.ds(base, b_per_w)], idx_v)

        @pl.loop(0, n_chunks)
        def _(c):
            off = c * W
            pltpu.async_copy(
                table_hbm.at[idx_v.at[pl.ds(off, W)]], rows_v, sem
            ).wait()
            pltpu.sync_copy(rows_v, out_hbm.at[pl.ds(base + off, W)])

    return k(table, idx_flat)


def kernel(idx, token_emb):
    B_, T_ = idx.shape
    B = B_ * T_
    idx_flat = idx.reshape(B).astype(jnp.int32)
    table_p = jnp.pad(token_emb, ((0, 0), (0, DP - D)))
    out = _gather_rows(table_p, idx_flat, B)
    return out[:, :D].reshape(B_, T_, D)
